# baseline (speedup 1.0000x reference)
import functools

import jax
import jax.numpy as jnp
from jax import lax
from jax.experimental import pallas as pl
from jax.experimental.pallas import tpu as pltpu
from jax.experimental.pallas import tpu_sc as plsc

T = 2048
D = 768
F = 3072
E = 64
C = 64
S = E * C
DUMP = S
SPAD = S + 16

NC = 2
NS = 16
NW = NC * NS
L = 16


def _router_body(x_ref, wg_ref, slot_ref):
    x = x_ref[...]
    wg = wg_ref[...]
    logits = jnp.dot(x, wg, preferred_element_type=jnp.float32)
    probs = jax.nn.softmax(logits, axis=-1)
    m = jnp.max(probs, axis=1, keepdims=True)
    col = lax.broadcasted_iota(jnp.int32, (T, E), 1)
    e_idx = jnp.min(jnp.where(probs == m, col, E), axis=1)
    onehot = (col == e_idx[:, None]).astype(jnp.int32)
    cum = onehot
    k = 1
    while k < T:
        cum = cum + jnp.concatenate(
            [jnp.zeros((k, E), jnp.int32), cum[: T - k]], axis=0)
        k *= 2
    pos = jnp.sum((cum - 1) * onehot, axis=1)
    valid = pos < C
    slot_ref[...] = jnp.where(valid, e_idx * C + pos, DUMP)


def _router(x, Wg):
    return pl.pallas_call(
        _router_body,
        out_shape=jax.ShapeDtypeStruct((T,), jnp.int32),
    )(x, Wg)


def _map_body(slot_hbm, map_hbm, slot_v, map_v):
    wid = lax.axis_index("s") * NC + lax.axis_index("c")

    @pl.when(wid == 0)
    def _():
        pltpu.sync_copy(slot_hbm, slot_v)

        def init(i, carry):
            map_v[pl.ds(i * L, L)] = (lax.iota(jnp.int32, L) + i * L) & (T - 1)
            return carry

        lax.fori_loop(0, SPAD // L, init, 0)

        def scat(i, carry):
            sv = slot_v[pl.ds(i * L, L)]
            tv = lax.iota(jnp.int32, L) + i * L
            plsc.store_scatter(map_v, [sv], tv)
            return carry

        lax.fori_loop(0, T // L, scat, 0)
        pltpu.sync_copy(map_v, map_hbm)


def _build_map(slot):
    mesh = plsc.VectorSubcoreMesh(core_axis_name="c", subcore_axis_name="s")
    return pl.kernel(
        _map_body,
        out_type=jax.ShapeDtypeStruct((SPAD,), jnp.int32),
        mesh=mesh,
        scratch_types=[
            pltpu.VMEM((T,), jnp.int32),
            pltpu.VMEM((SPAD,), jnp.int32),
        ],
        compiler_params=pltpu.CompilerParams(needs_layout_passes=False),
    )(slot)


def _dispatch_body(x_hbm, map_hbm, buf_hbm, idx_v, rows_v, sem):
    wid = lax.axis_index("s") * NC + lax.axis_index("c")
    base = wid * (S // NW)
    pltpu.sync_copy(map_hbm.at[pl.ds(base, S // NW)], idx_v)
    pltpu.async_copy(x_hbm.at[idx_v], rows_v, sem).wait()
    pltpu.sync_copy(rows_v, buf_hbm.at[pl.ds(base, S // NW)])


def _dispatch(x, tok_map):
    mesh = plsc.VectorSubcoreMesh(core_axis_name="c", subcore_axis_name="s")
    return pl.kernel(
        _dispatch_body,
        out_type=jax.ShapeDtypeStruct((S, D), jnp.float32),
        mesh=mesh,
        scratch_types=[
            pltpu.VMEM((S // NW,), jnp.int32),
            pltpu.VMEM((S // NW, D), jnp.float32),
            pltpu.SemaphoreType.DMA,
        ],
    )(x, tok_map)


def _ffn_body(buf_ref, w1_ref, b1_ref, w2_ref, b2_ref, out_ref):
    xb = buf_ref[...]
    h = jnp.dot(xb, w1_ref[0], preferred_element_type=jnp.float32)
    h = jax.nn.gelu(h + b1_ref[0])
    y = jnp.dot(h, w2_ref[0], preferred_element_type=jnp.float32)
    out_ref[...] = y + b2_ref[0]


def _ffn(buf, W1, b1, W2, b2):
    return pl.pallas_call(
        _ffn_body,
        grid=(E,),
        in_specs=[
            pl.BlockSpec((C, D), lambda e: (e, 0)),
            pl.BlockSpec((1, D, F), lambda e: (e, 0, 0)),
            pl.BlockSpec((1, 1, F), lambda e: (e, 0, 0)),
            pl.BlockSpec((1, F, D), lambda e: (e, 0, 0)),
            pl.BlockSpec((1, 1, D), lambda e: (e, 0, 0)),
        ],
        out_specs=pl.BlockSpec((C, D), lambda e: (e, 0)),
        out_shape=jax.ShapeDtypeStruct((S, D), jnp.float32),
        compiler_params=pltpu.CompilerParams(
            dimension_semantics=("parallel",),
        ),
    )(buf, W1, b1.reshape(E, 1, F), W2, b2.reshape(E, 1, D))


def _combine_body(y_hbm, slot_hbm, out_hbm, idx_v, rows_v, sem):
    wid = lax.axis_index("s") * NC + lax.axis_index("c")
    base = wid * (T // NW)
    pltpu.sync_copy(slot_hbm.at[pl.ds(base, T // NW)], idx_v)
    pltpu.async_copy(y_hbm.at[idx_v], rows_v, sem).wait()
    pltpu.sync_copy(rows_v, out_hbm.at[pl.ds(base, T // NW)])


def _combine(y_pad, slot):
    mesh = plsc.VectorSubcoreMesh(core_axis_name="c", subcore_axis_name="s")
    return pl.kernel(
        _combine_body,
        out_type=jax.ShapeDtypeStruct((T, D), jnp.float32),
        mesh=mesh,
        scratch_types=[
            pltpu.VMEM((T // NW,), jnp.int32),
            pltpu.VMEM((T // NW, D), jnp.float32),
            pltpu.SemaphoreType.DMA,
        ],
    )(y_pad, slot)


def kernel(x, Wg, W1, b1, W2, b2):
    slot = _router(x, Wg)
    tok_map = _build_map(slot)
    buf = _dispatch(x, tok_map)
    y_e = _ffn(buf, W1, b1, W2, b2)
    y_pad = jnp.concatenate(
        [y_e, jnp.zeros((SPAD - S, D), jnp.float32)], axis=0)
    return _combine(y_pad, slot)


# speedup vs baseline: 1.0215x; 1.0215x over previous
import functools

import jax
import jax.numpy as jnp
from jax import lax
from jax.experimental import pallas as pl
from jax.experimental.pallas import tpu as pltpu
from jax.experimental.pallas import tpu_sc as plsc

T = 2048
D = 768
F = 3072
E = 64
C = 64
S = E * C
DUMP = S
SPAD = S + C

NC = 2
NS = 16
NW = NC * NS
L = 16


def _router_body(x_ref, wg_ref, slot_ref):
    x = x_ref[...]
    wg = wg_ref[...]
    logits = jnp.dot(x, wg, preferred_element_type=jnp.float32)
    probs = jax.nn.softmax(logits, axis=-1)
    m = jnp.max(probs, axis=1, keepdims=True)
    col = lax.broadcasted_iota(jnp.int32, (T, E), 1)
    e_idx = jnp.min(jnp.where(probs == m, col, E), axis=1)
    onehot = (col == e_idx[:, None]).astype(jnp.int32)
    cum = onehot
    k = 1
    while k < T:
        cum = cum + jnp.concatenate(
            [jnp.zeros((k, E), jnp.int32), cum[: T - k]], axis=0)
        k *= 2
    pos = jnp.sum((cum - 1) * onehot, axis=1)
    valid = pos < C
    slot_ref[...] = jnp.where(valid, e_idx * C + pos, DUMP)


def _router(x, Wg):
    return pl.pallas_call(
        _router_body,
        out_shape=jax.ShapeDtypeStruct((T,), jnp.int32),
    )(x, Wg)


def _map_body(slot_hbm, map_hbm, slot_v, map_v):
    wid = lax.axis_index("s") * NC + lax.axis_index("c")

    @pl.when(wid == 0)
    def _():
        pltpu.sync_copy(slot_hbm, slot_v)

        def init(i, carry):
            map_v[pl.ds(i * L, L)] = (lax.iota(jnp.int32, L) + i * L) & (T - 1)
            return carry

        lax.fori_loop(0, SPAD // L, init, 0)

        def scat(i, carry):
            sv = slot_v[pl.ds(i * L, L)]
            tv = lax.iota(jnp.int32, L) + i * L
            plsc.store_scatter(map_v, [sv], tv)
            return carry

        lax.fori_loop(0, T // L, scat, 0)
        pltpu.sync_copy(map_v, map_hbm)


def _build_map(slot):
    mesh = plsc.VectorSubcoreMesh(core_axis_name="c", subcore_axis_name="s")
    return pl.kernel(
        _map_body,
        out_type=jax.ShapeDtypeStruct((SPAD,), jnp.int32),
        mesh=mesh,
        scratch_types=[
            pltpu.VMEM((T,), jnp.int32),
            pltpu.VMEM((SPAD,), jnp.int32),
        ],
        compiler_params=pltpu.CompilerParams(needs_layout_passes=False),
    )(slot)


def _dispatch_body(x_hbm, map_hbm, buf_hbm, idx_v, rows_v, sem):
    wid = lax.axis_index("s") * NC + lax.axis_index("c")
    base = wid * (S // NW)
    pltpu.sync_copy(map_hbm.at[pl.ds(base, S // NW)], idx_v)
    pltpu.async_copy(x_hbm.at[idx_v], rows_v, sem).wait()
    pltpu.sync_copy(rows_v, buf_hbm.at[pl.ds(base, S // NW)])


def _dispatch(x, tok_map):
    mesh = plsc.VectorSubcoreMesh(core_axis_name="c", subcore_axis_name="s")
    return pl.kernel(
        _dispatch_body,
        out_type=jax.ShapeDtypeStruct((S, D), jnp.float32),
        mesh=mesh,
        scratch_types=[
            pltpu.VMEM((S // NW,), jnp.int32),
            pltpu.VMEM((S // NW, D), jnp.float32),
            pltpu.SemaphoreType.DMA,
        ],
    )(x, tok_map)


def _ffn_body(buf_ref, w1_ref, b1_ref, w2_ref, b2_ref, out_ref):
    e = pl.program_id(0)

    @pl.when(e == 0)
    def _():
        out_ref[...] = jnp.zeros((C, D), jnp.float32)

    @pl.when(e > 0)
    def _():
        xb = buf_ref[...]
        h = jnp.dot(xb, w1_ref[0], preferred_element_type=jnp.float32)
        h = jax.nn.gelu(h + b1_ref[0])
        y = jnp.dot(h, w2_ref[0], preferred_element_type=jnp.float32)
        out_ref[...] = y + b2_ref[0]


def _ffn(buf, W1, b1, W2, b2):
    def widx(e):
        return (jnp.maximum(e - 1, 0), 0, 0)

    return pl.pallas_call(
        _ffn_body,
        grid=(E + 1,),
        in_specs=[
            pl.BlockSpec((C, D), lambda e: (jnp.maximum(e - 1, 0), 0)),
            pl.BlockSpec((1, D, F), widx),
            pl.BlockSpec((1, 1, F), widx),
            pl.BlockSpec((1, F, D), widx),
            pl.BlockSpec((1, 1, D), widx),
        ],
        out_specs=pl.BlockSpec(
            (C, D), lambda e: (jnp.where(e == 0, E, e - 1), 0)),
        out_shape=jax.ShapeDtypeStruct((SPAD, D), jnp.float32),
        compiler_params=pltpu.CompilerParams(
            dimension_semantics=("arbitrary",),
        ),
    )(buf, W1, b1.reshape(E, 1, F), W2, b2.reshape(E, 1, D))


def _combine_body(y_hbm, slot_hbm, out_hbm, idx_v, rows_v, sem):
    wid = lax.axis_index("s") * NC + lax.axis_index("c")
    base = wid * (T // NW)
    pltpu.sync_copy(slot_hbm.at[pl.ds(base, T // NW)], idx_v)
    pltpu.async_copy(y_hbm.at[idx_v], rows_v, sem).wait()
    pltpu.sync_copy(rows_v, out_hbm.at[pl.ds(base, T // NW)])


def _combine(y_pad, slot):
    mesh = plsc.VectorSubcoreMesh(core_axis_name="c", subcore_axis_name="s")
    return pl.kernel(
        _combine_body,
        out_type=jax.ShapeDtypeStruct((T, D), jnp.float32),
        mesh=mesh,
        scratch_types=[
            pltpu.VMEM((T // NW,), jnp.int32),
            pltpu.VMEM((T // NW, D), jnp.float32),
            pltpu.SemaphoreType.DMA,
        ],
    )(y_pad, slot)


def kernel(x, Wg, W1, b1, W2, b2):
    slot = _router(x, Wg)
    tok_map = _build_map(slot)
    buf = _dispatch(x, tok_map)
    y_pad = _ffn(buf, W1, b1, W2, b2)
    return _combine(y_pad, slot)


# speedup vs baseline: 1.0332x; 1.0114x over previous
import functools

import jax
import jax.numpy as jnp
from jax import lax
from jax.experimental import pallas as pl
from jax.experimental.pallas import tpu as pltpu
from jax.experimental.pallas import tpu_sc as plsc

T = 2048
D = 768
F = 3072
E = 64
C = 64
S = E * C
DUMP = S
SPAD = S + C

NC = 2
NS = 16
NW = NC * NS
L = 16


def _router_body(x_ref, wg_ref, slot_ref):
    x = x_ref[...]
    wg = wg_ref[...]
    logits = jnp.dot(x, wg, preferred_element_type=jnp.float32)
    probs = jax.nn.softmax(logits, axis=-1)
    m = jnp.max(probs, axis=1, keepdims=True)
    col = lax.broadcasted_iota(jnp.int32, (T, E), 1)
    e_idx = jnp.min(jnp.where(probs == m, col, E), axis=1)
    onehot = (col == e_idx[:, None]).astype(jnp.int32)
    cum = onehot
    k = 1
    while k < T:
        cum = cum + jnp.concatenate(
            [jnp.zeros((k, E), jnp.int32), cum[: T - k]], axis=0)
        k *= 2
    pos = jnp.sum((cum - 1) * onehot, axis=1)
    valid = pos < C
    slot_ref[...] = jnp.where(valid, e_idx * C + pos, DUMP)


def _router(x, Wg):
    return pl.pallas_call(
        _router_body,
        out_shape=jax.ShapeDtypeStruct((T,), jnp.int32),
    )(x, Wg)


def _dispatch_body(slot_hbm, x_hbm, buf_hbm, slot_v, map_v, map_sh, idx_v,
                   rows_v, sem):
    sid = lax.axis_index("s")
    wid = sid * NC + lax.axis_index("c")

    @pl.when(sid == 0)
    def _():
        pltpu.sync_copy(slot_hbm, slot_v)

        def init(i, carry):
            map_v[pl.ds(i * L, L)] = (lax.iota(jnp.int32, L) + i * L) & (T - 1)
            return carry

        lax.fori_loop(0, SPAD // L, init, 0)

        def scat(i, carry):
            sv = slot_v[pl.ds(i * L, L)]
            tv = lax.iota(jnp.int32, L) + i * L
            plsc.store_scatter(map_v, [sv], tv)
            return carry

        lax.fori_loop(0, T // L, scat, 0)
        pltpu.sync_copy(map_v, map_sh)

    plsc.subcore_barrier()
    base = wid * (S // NW)
    pltpu.sync_copy(map_sh.at[pl.ds(base, S // NW)], idx_v)
    pltpu.async_copy(x_hbm.at[idx_v], rows_v, sem).wait()
    pltpu.sync_copy(rows_v, buf_hbm.at[pl.ds(base, S // NW)])


def _dispatch(slot, x):
    mesh = plsc.VectorSubcoreMesh(core_axis_name="c", subcore_axis_name="s")
    return pl.kernel(
        _dispatch_body,
        out_type=jax.ShapeDtypeStruct((S, D), jnp.float32),
        mesh=mesh,
        scratch_types=[
            pltpu.VMEM((T,), jnp.int32),
            pltpu.VMEM((SPAD,), jnp.int32),
            pltpu.VMEM_SHARED((SPAD,), jnp.int32),
            pltpu.VMEM((S // NW,), jnp.int32),
            pltpu.VMEM((S // NW, D), jnp.float32),
            pltpu.SemaphoreType.DMA,
        ],
        compiler_params=pltpu.CompilerParams(needs_layout_passes=False),
    )(slot, x)


def _ffn_body(buf_ref, w1_ref, b1_ref, w2_ref, b2_ref, out_ref):
    e = pl.program_id(0)

    @pl.when(e == 0)
    def _():
        out_ref[...] = jnp.zeros((C, D), jnp.float32)

    @pl.when(e > 0)
    def _():
        xb = buf_ref[...]
        h = jnp.dot(xb, w1_ref[0], preferred_element_type=jnp.float32)
        h = jax.nn.gelu(h + b1_ref[0])
        y = jnp.dot(h, w2_ref[0], preferred_element_type=jnp.float32)
        out_ref[...] = y + b2_ref[0]


def _ffn(buf, W1, b1, W2, b2):
    def widx(e):
        return (jnp.maximum(e - 1, 0), 0, 0)

    return pl.pallas_call(
        _ffn_body,
        grid=(E + 1,),
        in_specs=[
            pl.BlockSpec((C, D), lambda e: (jnp.maximum(e - 1, 0), 0)),
            pl.BlockSpec((1, D, F), widx),
            pl.BlockSpec((1, 1, F), widx),
            pl.BlockSpec((1, F, D), widx),
            pl.BlockSpec((1, 1, D), widx),
        ],
        out_specs=pl.BlockSpec(
            (C, D), lambda e: (jnp.where(e == 0, E, e - 1), 0)),
        out_shape=jax.ShapeDtypeStruct((SPAD, D), jnp.float32),
        compiler_params=pltpu.CompilerParams(
            dimension_semantics=("arbitrary",),
        ),
    )(buf, W1, b1.reshape(E, 1, F), W2, b2.reshape(E, 1, D))


def _combine_body(y_hbm, slot_hbm, out_hbm, idx_v, rows_v, sem):
    wid = lax.axis_index("s") * NC + lax.axis_index("c")
    base = wid * (T // NW)
    pltpu.sync_copy(slot_hbm.at[pl.ds(base, T // NW)], idx_v)
    pltpu.async_copy(y_hbm.at[idx_v], rows_v, sem).wait()
    pltpu.sync_copy(rows_v, out_hbm.at[pl.ds(base, T // NW)])


def _combine(y_pad, slot):
    mesh = plsc.VectorSubcoreMesh(core_axis_name="c", subcore_axis_name="s")
    return pl.kernel(
        _combine_body,
        out_type=jax.ShapeDtypeStruct((T, D), jnp.float32),
        mesh=mesh,
        scratch_types=[
            pltpu.VMEM((T // NW,), jnp.int32),
            pltpu.VMEM((T // NW, D), jnp.float32),
            pltpu.SemaphoreType.DMA,
        ],
    )(y_pad, slot)


def kernel(x, Wg, W1, b1, W2, b2):
    slot = _router(x, Wg)
    buf = _dispatch(slot, x)
    y_pad = _ffn(buf, W1, b1, W2, b2)
    return _combine(y_pad, slot)


# speedup vs baseline: 1.0404x; 1.0069x over previous
import functools

import jax
import jax.numpy as jnp
from jax import lax
from jax.experimental import pallas as pl
from jax.experimental.pallas import tpu as pltpu
from jax.experimental.pallas import tpu_sc as plsc

T = 2048
D = 768
F = 3072
E = 64
C = 64
S = E * C
DUMP = S
SPAD = S + C

NC = 2
NS = 16
NW = NC * NS
L = 16


def _router_body(x_ref, wg_ref, slot_ref):
    x = x_ref[...]
    wg = wg_ref[...]
    logits = jnp.dot(x, wg, preferred_element_type=jnp.float32)
    probs = jax.nn.softmax(logits, axis=-1)
    m = jnp.max(probs, axis=1, keepdims=True)
    col = lax.broadcasted_iota(jnp.int32, (T, E), 1)
    e_idx = jnp.min(jnp.where(probs == m, col, E), axis=1)
    onehot = (col == e_idx[:, None]).astype(jnp.int32)
    cum = onehot
    k = 1
    while k < T:
        cum = cum + jnp.concatenate(
            [jnp.zeros((k, E), jnp.int32), cum[: T - k]], axis=0)
        k *= 2
    pos = jnp.sum((cum - 1) * onehot, axis=1)
    valid = pos < C
    slot_ref[...] = jnp.where(valid, e_idx * C + pos, DUMP)


def _router(x, Wg):
    return pl.pallas_call(
        _router_body,
        out_shape=jax.ShapeDtypeStruct((T,), jnp.int32),
    )(x, Wg)


def _dispatch_body(slot_hbm, x_hbm, buf_hbm, slot_v, map_v, map_sh, idx_v,
                   rows_v, sem):
    sid = lax.axis_index("s")
    wid = sid * NC + lax.axis_index("c")

    @pl.when(sid == 0)
    def _():
        pltpu.sync_copy(slot_hbm, slot_v)

        def init(i, carry):
            map_v[pl.ds(i * L, L)] = (lax.iota(jnp.int32, L) + i * L) & (T - 1)
            return carry

        lax.fori_loop(0, SPAD // L, init, 0)

        def scat(i, carry):
            sv = slot_v[pl.ds(i * L, L)]
            tv = lax.iota(jnp.int32, L) + i * L
            plsc.store_scatter(map_v, [sv], tv)
            return carry

        lax.fori_loop(0, T // L, scat, 0)
        pltpu.sync_copy(map_v, map_sh)

    plsc.subcore_barrier()
    base = wid * (S // NW)
    pltpu.sync_copy(map_sh.at[pl.ds(base, S // NW)], idx_v)
    pltpu.async_copy(x_hbm.at[idx_v], rows_v, sem).wait()
    pltpu.sync_copy(rows_v, buf_hbm.at[pl.ds(base, S // NW)])


def _dispatch(slot, x):
    mesh = plsc.VectorSubcoreMesh(core_axis_name="c", subcore_axis_name="s")
    return pl.kernel(
        _dispatch_body,
        out_type=jax.ShapeDtypeStruct((S, D), jnp.float32),
        mesh=mesh,
        scratch_types=[
            pltpu.VMEM((T,), jnp.int32),
            pltpu.VMEM((SPAD,), jnp.int32),
            pltpu.VMEM_SHARED((SPAD,), jnp.int32),
            pltpu.VMEM((S // NW,), jnp.int32),
            pltpu.VMEM((S // NW, D), jnp.float32),
            pltpu.SemaphoreType.DMA,
        ],
        compiler_params=pltpu.CompilerParams(needs_layout_passes=False),
    )(slot, x)


def _ffn_body(buf_ref, w1a_ref, w1b_ref, b1a_ref, b1b_ref, w2a_ref, w2b_ref,
              b2_ref, out_ref):
    e = pl.program_id(0)

    @pl.when(e == 0)
    def _():
        out_ref[...] = jnp.zeros((C, D), jnp.float32)

    @pl.when(e > 0)
    def _():
        xb = buf_ref[...]
        ha = jnp.dot(xb, w1a_ref[0], preferred_element_type=jnp.float32)
        ha = jax.nn.gelu(ha + b1a_ref[0])
        hb = jnp.dot(xb, w1b_ref[0], preferred_element_type=jnp.float32)
        hb = jax.nn.gelu(hb + b1b_ref[0])
        y = (jnp.dot(ha, w2a_ref[0], preferred_element_type=jnp.float32)
             + jnp.dot(hb, w2b_ref[0], preferred_element_type=jnp.float32))
        out_ref[...] = y + b2_ref[0]


def _ffn(buf, W1, b1, W2, b2):
    def wa(e):
        return (jnp.maximum(e - 1, 0), 0, 0)

    def wb(e):
        return (jnp.maximum(e - 1, 0), 0, 1)

    def w2b(e):
        return (jnp.maximum(e - 1, 0), 1, 0)

    return pl.pallas_call(
        _ffn_body,
        grid=(E + 1,),
        in_specs=[
            pl.BlockSpec((C, D), lambda e: (jnp.maximum(e - 1, 0), 0)),
            pl.BlockSpec((1, D, F // 2), wa),
            pl.BlockSpec((1, D, F // 2), wb),
            pl.BlockSpec((1, 1, F // 2), wa),
            pl.BlockSpec((1, 1, F // 2), wb),
            pl.BlockSpec((1, F // 2, D), wa),
            pl.BlockSpec((1, F // 2, D), w2b),
            pl.BlockSpec((1, 1, D), wa),
        ],
        out_specs=pl.BlockSpec(
            (C, D), lambda e: (jnp.where(e == 0, E, e - 1), 0)),
        out_shape=jax.ShapeDtypeStruct((SPAD, D), jnp.float32),
        compiler_params=pltpu.CompilerParams(
            dimension_semantics=("arbitrary",),
        ),
    )(buf, W1, W1, b1.reshape(E, 1, F), b1.reshape(E, 1, F),
      W2, W2, b2.reshape(E, 1, D))


def _combine_body(y_hbm, slot_hbm, out_hbm, idx_v, rows_v, sem):
    wid = lax.axis_index("s") * NC + lax.axis_index("c")
    base = wid * (T // NW)
    pltpu.sync_copy(slot_hbm.at[pl.ds(base, T // NW)], idx_v)
    pltpu.async_copy(y_hbm.at[idx_v], rows_v, sem).wait()
    pltpu.sync_copy(rows_v, out_hbm.at[pl.ds(base, T // NW)])


def _combine(y_pad, slot):
    mesh = plsc.VectorSubcoreMesh(core_axis_name="c", subcore_axis_name="s")
    return pl.kernel(
        _combine_body,
        out_type=jax.ShapeDtypeStruct((T, D), jnp.float32),
        mesh=mesh,
        scratch_types=[
            pltpu.VMEM((T // NW,), jnp.int32),
            pltpu.VMEM((T // NW, D), jnp.float32),
            pltpu.SemaphoreType.DMA,
        ],
    )(y_pad, slot)


def kernel(x, Wg, W1, b1, W2, b2):
    slot = _router(x, Wg)
    buf = _dispatch(slot, x)
    y_pad = _ffn(buf, W1, b1, W2, b2)
    return _combine(y_pad, slot)


# speedup vs baseline: 1.0622x; 1.0210x over previous
import functools

import jax
import jax.numpy as jnp
from jax import lax
from jax.experimental import pallas as pl
from jax.experimental.pallas import tpu as pltpu
from jax.experimental.pallas import tpu_sc as plsc

T = 2048
D = 768
F = 3072
E = 64
C = 64
S = E * C
DUMP = S
SPAD = S + C

NC = 2
NS = 16
NW = NC * NS
L = 16


def _router_body(x_ref, wg_ref, slot_ref):
    x = x_ref[...]
    wg = wg_ref[...]
    logits = jnp.dot(x, wg, preferred_element_type=jnp.float32)
    probs = jax.nn.softmax(logits, axis=-1)
    m = jnp.max(probs, axis=1, keepdims=True)
    col = lax.broadcasted_iota(jnp.int32, (T, E), 1)
    e_idx = jnp.min(jnp.where(probs == m, col, E), axis=1)
    onehot = (col == e_idx[:, None]).astype(jnp.int32)
    cum = onehot
    k = 1
    while k < T:
        cum = cum + jnp.concatenate(
            [jnp.zeros((k, E), jnp.int32), cum[: T - k]], axis=0)
        k *= 2
    pos = jnp.sum((cum - 1) * onehot, axis=1)
    valid = pos < C
    row = lax.iota(jnp.int32, T) & (C - 1)
    slot_ref[...] = jnp.where(valid, e_idx * C + pos, S + row)


def _router(x, Wg):
    return pl.pallas_call(
        _router_body,
        out_shape=jax.ShapeDtypeStruct((T,), jnp.int32),
    )(x, Wg)


TPW = T // NW


def _dispatch_body(slot_hbm, x_hbm, buf_hbm, sidx_v, rows_v, sem):
    wid = lax.axis_index("s") * NC + lax.axis_index("c")
    base = wid * TPW
    pltpu.sync_copy(slot_hbm.at[pl.ds(base, TPW)], sidx_v)
    pltpu.sync_copy(x_hbm.at[pl.ds(base, TPW)], rows_v)
    pltpu.async_copy(rows_v, buf_hbm.at[sidx_v], sem).wait()


def _dispatch(slot, x):
    mesh = plsc.VectorSubcoreMesh(core_axis_name="c", subcore_axis_name="s")
    return pl.kernel(
        _dispatch_body,
        out_type=jax.ShapeDtypeStruct((SPAD, D), jnp.float32),
        mesh=mesh,
        scratch_types=[
            pltpu.VMEM((TPW,), jnp.int32),
            pltpu.VMEM((TPW, D), jnp.float32),
            pltpu.SemaphoreType.DMA,
        ],
    )(slot, x)


def _ffn_body(buf_ref, w1a_ref, w1b_ref, b1a_ref, b1b_ref, w2a_ref, w2b_ref,
              b2_ref, out_ref):
    e = pl.program_id(0)

    @pl.when(e == 0)
    def _():
        out_ref[...] = jnp.zeros((C, D), jnp.float32)

    @pl.when(e > 0)
    def _():
        xb = buf_ref[...]
        ha = jnp.dot(xb, w1a_ref[0], preferred_element_type=jnp.float32)
        ha = jax.nn.gelu(ha + b1a_ref[0])
        hb = jnp.dot(xb, w1b_ref[0], preferred_element_type=jnp.float32)
        hb = jax.nn.gelu(hb + b1b_ref[0])
        y = (jnp.dot(ha, w2a_ref[0], preferred_element_type=jnp.float32)
             + jnp.dot(hb, w2b_ref[0], preferred_element_type=jnp.float32))
        out_ref[...] = y + b2_ref[0]


def _ffn(buf, W1, b1, W2, b2):
    def wa(e):
        return (jnp.maximum(e - 1, 0), 0, 0)

    def wb(e):
        return (jnp.maximum(e - 1, 0), 0, 1)

    def w2b(e):
        return (jnp.maximum(e - 1, 0), 1, 0)

    return pl.pallas_call(
        _ffn_body,
        grid=(E + 1,),
        in_specs=[
            pl.BlockSpec((C, D), lambda e: (jnp.maximum(e - 1, 0), 0)),
            pl.BlockSpec((1, D, F // 2), wa),
            pl.BlockSpec((1, D, F // 2), wb),
            pl.BlockSpec((1, 1, F // 2), wa),
            pl.BlockSpec((1, 1, F // 2), wb),
            pl.BlockSpec((1, F // 2, D), wa),
            pl.BlockSpec((1, F // 2, D), w2b),
            pl.BlockSpec((1, 1, D), wa),
        ],
        out_specs=pl.BlockSpec(
            (C, D), lambda e: (jnp.where(e == 0, E, e - 1), 0)),
        out_shape=jax.ShapeDtypeStruct((SPAD, D), jnp.float32),
        compiler_params=pltpu.CompilerParams(
            dimension_semantics=("arbitrary",),
        ),
    )(buf, W1, W1, b1.reshape(E, 1, F), b1.reshape(E, 1, F),
      W2, W2, b2.reshape(E, 1, D))


def _combine_body(y_hbm, slot_hbm, out_hbm, idx_v, rows_v, sem):
    wid = lax.axis_index("s") * NC + lax.axis_index("c")
    base = wid * (T // NW)
    pltpu.sync_copy(slot_hbm.at[pl.ds(base, T // NW)], idx_v)
    pltpu.async_copy(y_hbm.at[idx_v], rows_v, sem).wait()
    pltpu.sync_copy(rows_v, out_hbm.at[pl.ds(base, T // NW)])


def _combine(y_pad, slot):
    mesh = plsc.VectorSubcoreMesh(core_axis_name="c", subcore_axis_name="s")
    return pl.kernel(
        _combine_body,
        out_type=jax.ShapeDtypeStruct((T, D), jnp.float32),
        mesh=mesh,
        scratch_types=[
            pltpu.VMEM((T // NW,), jnp.int32),
            pltpu.VMEM((T // NW, D), jnp.float32),
            pltpu.SemaphoreType.DMA,
        ],
    )(y_pad, slot)


def kernel(x, Wg, W1, b1, W2, b2):
    slot = _router(x, Wg)
    buf = _dispatch(slot, x)
    y_pad = _ffn(buf, W1, b1, W2, b2)
    return _combine(y_pad, slot)
